# baseline (speedup 1.0000x reference)
import functools

import jax
import jax.numpy as jnp
from jax import lax
from jax.experimental import pallas as pl
from jax.experimental.pallas import tpu as pltpu
from jax.experimental.pallas import tpu_sc as plsc

_LAMBDA = 0.05
_BATCH = 16384
_DIM = 64
_CHUNK = 128
_LANES = 16


def _sc_body(ent1, ent2, rel1, rel2, pos_h, pos_t, pos_r, neg_h, neg_t,
             neg_r, p_out, n_out, sq_out,
             idx_h, idx_t, idx_r, e1h, e2h, e1t, e2t, r1b, r2b,
             score_buf, acc_buf, sem0, sem1, sem2, sem3, sem4, sem5):
  nc = 2
  wid = lax.axis_index("s") * nc + lax.axis_index("c")
  per_w = _BATCH // 32
  base = wid * per_w
  n_chunks = per_w // _CHUNK

  acc = jnp.zeros((_LANES,), jnp.float32)

  for side in range(2):
    h_hbm = pos_h if side == 0 else neg_h
    t_hbm = pos_t if side == 0 else neg_t
    r_hbm = pos_r if side == 0 else neg_r
    s_out = p_out if side == 0 else n_out
    for k in range(n_chunks):
      b0 = base + k * _CHUNK
      pltpu.sync_copy(h_hbm.at[pl.ds(b0, _CHUNK)], idx_h)
      pltpu.sync_copy(t_hbm.at[pl.ds(b0, _CHUNK)], idx_t)
      pltpu.sync_copy(r_hbm.at[pl.ds(b0, _CHUNK)], idx_r)
      cps = [
          pltpu.async_copy(ent1.at[idx_h], e1h, sem0),
          pltpu.async_copy(ent2.at[idx_h], e2h, sem1),
          pltpu.async_copy(ent1.at[idx_t], e1t, sem2),
          pltpu.async_copy(ent2.at[idx_t], e2t, sem3),
          pltpu.async_copy(rel1.at[idx_r], r1b, sem4),
          pltpu.async_copy(rel2.at[idx_r], r2b, sem5),
      ]
      for cp in cps:
        cp.wait()

      def elem(e, acc):
        s = jnp.zeros((_LANES,), jnp.float32)
        for c in range(_DIM // _LANES):
          sl = pl.ds(c * _LANES, _LANES)
          v1h = e1h[e, sl]
          v2h = e2h[e, sl]
          v1t = e1t[e, sl]
          v2t = e2t[e, sl]
          vr1 = r1b[e, sl]
          vr2 = r2b[e, sl]
          acc = acc + v1h * v1h + v2h * v2h + v1t * v1t
          acc = acc + v2t * v2t + vr1 * vr1 + vr2 * vr2
          s = s + (v1h * v1t + v2h * v2t) * vr1
          s = s + (v1h * v2t - v2h * v1t) * vr2
        score_buf[e, :] = s
        return acc

      acc = lax.fori_loop(0, _CHUNK, elem, acc)
      pltpu.sync_copy(score_buf, s_out.at[pl.ds(b0, _CHUNK)])

  acc_buf[...] = acc
  pltpu.sync_copy(acc_buf, sq_out.at[wid])


def _sc_call(ent1, ent2, rel1, rel2, pos_h, pos_t, pos_r, neg_h, neg_t,
             neg_r):
  mesh = plsc.VectorSubcoreMesh(core_axis_name="c", subcore_axis_name="s")
  f = pl.kernel(
      _sc_body,
      compiler_params=pltpu.CompilerParams(use_tc_tiling_on_sc=False),
      out_type=(
          jax.ShapeDtypeStruct((_BATCH, _LANES), jnp.float32),
          jax.ShapeDtypeStruct((_BATCH, _LANES), jnp.float32),
          jax.ShapeDtypeStruct((32, _LANES), jnp.float32),
      ),
      mesh=mesh,
      scratch_types=[
          pltpu.VMEM((_CHUNK,), jnp.int32),
          pltpu.VMEM((_CHUNK,), jnp.int32),
          pltpu.VMEM((_CHUNK,), jnp.int32),
          pltpu.VMEM((_CHUNK, _DIM), jnp.float32),
          pltpu.VMEM((_CHUNK, _DIM), jnp.float32),
          pltpu.VMEM((_CHUNK, _DIM), jnp.float32),
          pltpu.VMEM((_CHUNK, _DIM), jnp.float32),
          pltpu.VMEM((_CHUNK, _DIM), jnp.float32),
          pltpu.VMEM((_CHUNK, _DIM), jnp.float32),
          pltpu.VMEM((_CHUNK, _LANES), jnp.float32),
          pltpu.VMEM((_LANES,), jnp.float32),
          pltpu.SemaphoreType.DMA,
          pltpu.SemaphoreType.DMA,
          pltpu.SemaphoreType.DMA,
          pltpu.SemaphoreType.DMA,
          pltpu.SemaphoreType.DMA,
          pltpu.SemaphoreType.DMA,
      ],
  )
  return f(ent1, ent2, rel1, rel2, pos_h, pos_t, pos_r, neg_h, neg_t,
           neg_r)


def _tc_body(p_ref, n_ref, sq_ref, o_ref):
  p = jnp.sum(p_ref[...], axis=1)
  n = jnp.sum(n_ref[...], axis=1)

  def softplus(x):
    return jnp.maximum(x, 0.0) + jnp.log1p(jnp.exp(-jnp.abs(x)))

  loss = jnp.sum(softplus(-p)) + jnp.sum(softplus(n))
  loss = loss + _LAMBDA * jnp.sum(sq_ref[...])
  o_ref[0, 0] = loss


def _tc_call(p_part, n_part, sq_part):
  return pl.pallas_call(
      _tc_body,
      out_shape=jax.ShapeDtypeStruct((1, 1), jnp.float32),
      out_specs=pl.BlockSpec(memory_space=pltpu.SMEM),
  )(p_part, n_part, sq_part)


@jax.jit
def kernel(pos_h, pos_t, pos_r, neg_h, neg_t, neg_r, ent1_embeddings,
           ent2_embeddings, rel1_embeddings, rel2_embeddings):
  p_part, n_part, sq_part = _sc_call(
      ent1_embeddings, ent2_embeddings, rel1_embeddings, rel2_embeddings,
      pos_h.astype(jnp.int32), pos_t.astype(jnp.int32),
      pos_r.astype(jnp.int32), neg_h.astype(jnp.int32),
      neg_t.astype(jnp.int32), neg_r.astype(jnp.int32))
  out = _tc_call(p_part, n_part, sq_part)
  return out[0, 0]


# speedup vs baseline: 1.9870x; 1.9870x over previous
import jax
import jax.numpy as jnp
from jax import lax
from jax.experimental import pallas as pl
from jax.experimental.pallas import tpu as pltpu
from jax.experimental.pallas import tpu_sc as plsc

_LAMBDA = 0.05
_BATCH = 16384
_DIM = 64
_CHUNK = 128
_LANES = 16


def _sc_body(ent, rel, pos_h, pos_t, pos_r, neg_h, neg_t, neg_r,
             sc_out, sq_out,
             idx_h, idx_t, idx_r, eh, et, rb, score_buf, acc_buf,
             sem0, sem1, sem2):
  nc = 2
  wid = lax.axis_index("s") * nc + lax.axis_index("c")
  per_w = _BATCH // 32
  base = wid * per_w
  n_chunks = per_w // _CHUNK

  acc = jnp.zeros((_LANES,), jnp.float32)

  for side in range(2):
    h_hbm = pos_h if side == 0 else neg_h
    t_hbm = pos_t if side == 0 else neg_t
    r_hbm = pos_r if side == 0 else neg_r
    s_out = sc_out
    for k in range(n_chunks):
      b0 = base + k * _CHUNK
      pltpu.sync_copy(h_hbm.at[pl.ds(b0, _CHUNK)], idx_h)
      pltpu.sync_copy(t_hbm.at[pl.ds(b0, _CHUNK)], idx_t)
      pltpu.sync_copy(r_hbm.at[pl.ds(b0, _CHUNK)], idx_r)
      cps = [
          pltpu.async_copy(ent.at[idx_h], eh, sem0),
          pltpu.async_copy(ent.at[idx_t], et, sem1),
          pltpu.async_copy(rel.at[idx_r], rb, sem2),
      ]
      for cp in cps:
        cp.wait()

      sgn = -1.0 if side == 0 else 1.0

      def group8(g8, acc):
        for j in range(8):
          e = g8 * 8 + j
          s = jnp.zeros((_LANES,), jnp.float32)
          for c in range(_DIM // _LANES):
            lo = pl.ds(c * _LANES, _LANES)
            hi = pl.ds(_DIM + c * _LANES, _LANES)
            v1h = eh[e, lo]
            v2h = eh[e, hi]
            v1t = et[e, lo]
            v2t = et[e, hi]
            vr1 = rb[e, lo]
            vr2 = rb[e, hi]
            acc = acc + v1h * v1h + v2h * v2h + v1t * v1t
            acc = acc + v2t * v2t + vr1 * vr1 + vr2 * vr2
            s = s + (v1h * v1t + v2h * v2t) * vr1
            s = s + (v1h * v2t - v2h * v1t) * vr2
          score_buf[g8, pl.ds(j * _LANES, _LANES)] = sgn * s
        return acc

      acc = lax.fori_loop(0, _CHUNK // 8, group8, acc)
      row0 = (side * (_BATCH // 8) + wid * (per_w // 8) +
              k * (_CHUNK // 8))
      pltpu.sync_copy(score_buf, s_out.at[pl.ds(row0, _CHUNK // 8)])

  acc_buf[...] = acc
  pltpu.sync_copy(acc_buf, sq_out.at[wid])


def _sc_call(ent, rel, pos_h, pos_t, pos_r, neg_h, neg_t, neg_r):
  mesh = plsc.VectorSubcoreMesh(core_axis_name="c", subcore_axis_name="s")
  f = pl.kernel(
      _sc_body,
      out_type=(
          jax.ShapeDtypeStruct((2 * _BATCH // 8, 8 * _LANES), jnp.float32),
          jax.ShapeDtypeStruct((32, _LANES), jnp.float32),
      ),
      mesh=mesh,
      scratch_types=[
          pltpu.VMEM((_CHUNK,), jnp.int32),
          pltpu.VMEM((_CHUNK,), jnp.int32),
          pltpu.VMEM((_CHUNK,), jnp.int32),
          pltpu.VMEM((_CHUNK, 2 * _DIM), jnp.float32),
          pltpu.VMEM((_CHUNK, 2 * _DIM), jnp.float32),
          pltpu.VMEM((_CHUNK, 2 * _DIM), jnp.float32),
          pltpu.VMEM((_CHUNK // 8, 8 * _LANES), jnp.float32),
          pltpu.VMEM((_LANES,), jnp.float32),
          pltpu.SemaphoreType.DMA,
          pltpu.SemaphoreType.DMA,
          pltpu.SemaphoreType.DMA,
      ],
  )
  return f(ent, rel, pos_h, pos_t, pos_r, neg_h, neg_t, neg_r)


def _tc_body(s_ref, sq_ref, o_ref):
  x = s_ref[...]
  gsel = (lax.broadcasted_iota(jnp.int32, (8, 8 * _LANES), 1) // _LANES ==
          lax.broadcasted_iota(jnp.int32, (8, 8 * _LANES), 0)
          ).astype(jnp.float32)
  z = lax.dot_general(gsel, x, (((1,), (1,)), ((), ())),
                      preferred_element_type=jnp.float32)
  sp = jnp.maximum(z, 0.0) + jnp.log1p(jnp.exp(-jnp.abs(z)))
  loss = jnp.sum(sp) + _LAMBDA * jnp.sum(sq_ref[...])
  o_ref[0, 0] = loss


def _tc_call(scores, sq_part):
  return pl.pallas_call(
      _tc_body,
      out_shape=jax.ShapeDtypeStruct((1, 1), jnp.float32),
      out_specs=pl.BlockSpec(memory_space=pltpu.SMEM),
  )(scores, sq_part)


def _tr_body(e1t_ref, e2t_ref, o_ref):
  ident = (lax.broadcasted_iota(jnp.int32, (_DIM, _DIM), 0) ==
           lax.broadcasted_iota(jnp.int32, (_DIM, _DIM), 1)
           ).astype(jnp.float32)
  dn = (((0,), (0,)), ((), ()))
  a = lax.dot_general(e1t_ref[...], ident, dn,
                      preferred_element_type=jnp.float32)
  b = lax.dot_general(e2t_ref[...], ident, dn,
                      preferred_element_type=jnp.float32)
  o_ref[...] = jnp.concatenate([a, b], axis=1)


def _transpose_concat(a_t, b_t, n_rows, blk):
  grid = (n_rows + blk - 1) // blk
  return pl.pallas_call(
      _tr_body,
      grid=(grid,),
      in_specs=[
          pl.BlockSpec((_DIM, blk), lambda i: (0, i)),
          pl.BlockSpec((_DIM, blk), lambda i: (0, i)),
      ],
      out_specs=pl.BlockSpec((blk, 2 * _DIM), lambda i: (i, 0)),
      out_shape=jax.ShapeDtypeStruct((n_rows, 2 * _DIM), jnp.float32),
  )(a_t, b_t)


@jax.jit
def kernel(pos_h, pos_t, pos_r, neg_h, neg_t, neg_r, ent1_embeddings,
           ent2_embeddings, rel1_embeddings, rel2_embeddings):
  ent = _transpose_concat(ent1_embeddings.T, ent2_embeddings.T,
                          1000000, 4096)
  rel = _transpose_concat(rel1_embeddings.T, rel2_embeddings.T,
                          1000, 1024)
  scores, sq_part = _sc_call(
      ent, rel,
      pos_h.astype(jnp.int32), pos_t.astype(jnp.int32),
      pos_r.astype(jnp.int32), neg_h.astype(jnp.int32),
      neg_t.astype(jnp.int32), neg_r.astype(jnp.int32))
  out = _tc_call(scores, sq_part)
  return out[0, 0]


# speedup vs baseline: 2.4705x; 1.2433x over previous
import jax
import jax.numpy as jnp
from jax import lax
from jax.experimental import pallas as pl
from jax.experimental.pallas import tpu as pltpu
from jax.experimental.pallas import tpu_sc as plsc

_LAMBDA = 0.05
_BATCH = 16384
_DIM = 64
_CHUNK = 128
_LANES = 16


def _sc_body(ent, rel, pos_h, pos_t, pos_r, neg_h, neg_t, neg_r,
             sc_out, sq_out,
             idx_h0, idx_t0, idx_r0, idx_h1, idx_t1, idx_r1,
             eh0, et0, rb0, eh1, et1, rb1, score_buf, acc_buf,
             sem0, sem1, sem2, sem3, sem4, sem5):
  nc = 2
  wid = lax.axis_index("s") * nc + lax.axis_index("c")
  per_w = _BATCH // 32
  base = wid * per_w
  n_chunks = per_w // _CHUNK

  idx_bufs = ((idx_h0, idx_t0, idx_r0), (idx_h1, idx_t1, idx_r1))
  row_bufs = ((eh0, et0, rb0), (eh1, et1, rb1))
  sems = ((sem0, sem1, sem2), (sem3, sem4, sem5))
  phases = [(side, k) for side in range(2) for k in range(n_chunks)]

  def fire(i):
    side, k = phases[i]
    h_hbm = pos_h if side == 0 else neg_h
    t_hbm = pos_t if side == 0 else neg_t
    r_hbm = pos_r if side == 0 else neg_r
    ih, it, ir = idx_bufs[i % 2]
    eh, et, rb = row_bufs[i % 2]
    sm = sems[i % 2]
    b0 = base + k * _CHUNK
    pltpu.sync_copy(h_hbm.at[pl.ds(b0, _CHUNK)], ih)
    pltpu.sync_copy(t_hbm.at[pl.ds(b0, _CHUNK)], it)
    pltpu.sync_copy(r_hbm.at[pl.ds(b0, _CHUNK)], ir)
    return [
        pltpu.async_copy(ent.at[ih], eh, sm[0]),
        pltpu.async_copy(ent.at[it], et, sm[1]),
        pltpu.async_copy(rel.at[ir], rb, sm[2]),
    ]

  acc = jnp.zeros((_LANES,), jnp.float32)
  cps = fire(0)
  for i, (side, k) in enumerate(phases):
    for cp in cps:
      cp.wait()
    if i + 1 < len(phases):
      nxt = fire(i + 1)
    eh, et, rb = row_bufs[i % 2]
    sgn = -1.0 if side == 0 else 1.0

    def group8(g8, acc):
      for j in range(8):
        e = g8 * 8 + j
        s = jnp.zeros((_LANES,), jnp.float32)
        for c in range(_DIM // _LANES):
          lo = pl.ds(c * _LANES, _LANES)
          hi = pl.ds(_DIM + c * _LANES, _LANES)
          v1h = eh[e, lo]
          v2h = eh[e, hi]
          v1t = et[e, lo]
          v2t = et[e, hi]
          vr1 = rb[e, lo]
          vr2 = rb[e, hi]
          acc = acc + v1h * v1h + v2h * v2h + v1t * v1t
          acc = acc + v2t * v2t + vr1 * vr1 + vr2 * vr2
          s = s + (v1h * v1t + v2h * v2t) * vr1
          s = s + (v1h * v2t - v2h * v1t) * vr2
        score_buf[g8, pl.ds(j * _LANES, _LANES)] = sgn * s
      return acc

    acc = lax.fori_loop(0, _CHUNK // 8, group8, acc)
    row0 = (side * (_BATCH // 8) + wid * (per_w // 8) +
            k * (_CHUNK // 8))
    pltpu.sync_copy(score_buf, sc_out.at[pl.ds(row0, _CHUNK // 8)])
    if i + 1 < len(phases):
      cps = nxt

  acc_buf[...] = acc
  pltpu.sync_copy(acc_buf, sq_out.at[wid])


def _sc_call(ent, rel, pos_h, pos_t, pos_r, neg_h, neg_t, neg_r):
  mesh = plsc.VectorSubcoreMesh(core_axis_name="c", subcore_axis_name="s")
  f = pl.kernel(
      _sc_body,
      out_type=(
          jax.ShapeDtypeStruct((2 * _BATCH // 8, 8 * _LANES), jnp.float32),
          jax.ShapeDtypeStruct((32, _LANES), jnp.float32),
      ),
      mesh=mesh,
      scratch_types=[
          pltpu.VMEM((_CHUNK,), jnp.int32),
          pltpu.VMEM((_CHUNK,), jnp.int32),
          pltpu.VMEM((_CHUNK,), jnp.int32),
          pltpu.VMEM((_CHUNK,), jnp.int32),
          pltpu.VMEM((_CHUNK,), jnp.int32),
          pltpu.VMEM((_CHUNK,), jnp.int32),
          pltpu.VMEM((_CHUNK, 2 * _DIM), jnp.float32),
          pltpu.VMEM((_CHUNK, 2 * _DIM), jnp.float32),
          pltpu.VMEM((_CHUNK, 2 * _DIM), jnp.float32),
          pltpu.VMEM((_CHUNK, 2 * _DIM), jnp.float32),
          pltpu.VMEM((_CHUNK, 2 * _DIM), jnp.float32),
          pltpu.VMEM((_CHUNK, 2 * _DIM), jnp.float32),
          pltpu.VMEM((_CHUNK // 8, 8 * _LANES), jnp.float32),
          pltpu.VMEM((_LANES,), jnp.float32),
          pltpu.SemaphoreType.DMA,
          pltpu.SemaphoreType.DMA,
          pltpu.SemaphoreType.DMA,
          pltpu.SemaphoreType.DMA,
          pltpu.SemaphoreType.DMA,
          pltpu.SemaphoreType.DMA,
      ],
  )
  return f(ent, rel, pos_h, pos_t, pos_r, neg_h, neg_t, neg_r)


def _tc_body(s_ref, sq_ref, o_ref):
  x = s_ref[...]
  gsel = (lax.broadcasted_iota(jnp.int32, (8, 8 * _LANES), 1) // _LANES ==
          lax.broadcasted_iota(jnp.int32, (8, 8 * _LANES), 0)
          ).astype(jnp.float32)
  z = lax.dot_general(gsel, x, (((1,), (1,)), ((), ())),
                      preferred_element_type=jnp.float32)
  sp = jnp.maximum(z, 0.0) + jnp.log1p(jnp.exp(-jnp.abs(z)))
  loss = jnp.sum(sp) + _LAMBDA * jnp.sum(sq_ref[...])
  o_ref[0, 0] = loss


def _tc_call(scores, sq_part):
  return pl.pallas_call(
      _tc_body,
      out_shape=jax.ShapeDtypeStruct((1, 1), jnp.float32),
      out_specs=pl.BlockSpec(memory_space=pltpu.SMEM),
  )(scores, sq_part)


def _tr_body(e1t_ref, e2t_ref, o_ref):
  ident = (lax.broadcasted_iota(jnp.int32, (_DIM, _DIM), 0) ==
           lax.broadcasted_iota(jnp.int32, (_DIM, _DIM), 1)
           ).astype(jnp.float32)
  dn = (((0,), (0,)), ((), ()))
  a = lax.dot_general(e1t_ref[...], ident, dn,
                      preferred_element_type=jnp.float32)
  b = lax.dot_general(e2t_ref[...], ident, dn,
                      preferred_element_type=jnp.float32)
  o_ref[...] = jnp.concatenate([a, b], axis=1)


def _transpose_concat(a_t, b_t, n_rows, blk):
  grid = (n_rows + blk - 1) // blk
  return pl.pallas_call(
      _tr_body,
      grid=(grid,),
      in_specs=[
          pl.BlockSpec((_DIM, blk), lambda i: (0, i)),
          pl.BlockSpec((_DIM, blk), lambda i: (0, i)),
      ],
      out_specs=pl.BlockSpec((blk, 2 * _DIM), lambda i: (i, 0)),
      out_shape=jax.ShapeDtypeStruct((n_rows, 2 * _DIM), jnp.float32),
  )(a_t, b_t)


@jax.jit
def kernel(pos_h, pos_t, pos_r, neg_h, neg_t, neg_r, ent1_embeddings,
           ent2_embeddings, rel1_embeddings, rel2_embeddings):
  ent = _transpose_concat(ent1_embeddings.T, ent2_embeddings.T,
                          1000000, 16384)
  rel = _transpose_concat(rel1_embeddings.T, rel2_embeddings.T,
                          1000, 1024)
  scores, sq_part = _sc_call(
      ent, rel,
      pos_h.astype(jnp.int32), pos_t.astype(jnp.int32),
      pos_r.astype(jnp.int32), neg_h.astype(jnp.int32),
      neg_t.astype(jnp.int32), neg_r.astype(jnp.int32))
  out = _tc_call(scores, sq_part)
  return out[0, 0]


# speedup vs baseline: 2.8921x; 1.1706x over previous
import jax
import jax.numpy as jnp
from jax import lax
from jax.experimental import pallas as pl
from jax.experimental.pallas import tpu as pltpu
from jax.experimental.pallas import tpu_sc as plsc

_LAMBDA = 0.05
_BATCH = 16384
_DIM = 64
_CHUNK = 128
_LANES = 16


def _sc_body(ent, rel, pos_h, pos_t, pos_r, neg_h, neg_t, neg_r,
             sc_out, sq_out,
             idx_h0, idx_t0, idx_r0, idx_h1, idx_t1, idx_r1,
             eh0, et0, rb0, eh1, et1, rb1, score_buf, acc_buf,
             sem0, sem1, sem2, sem3, sem4, sem5):
  nc = 2
  wid = lax.axis_index("s") * nc + lax.axis_index("c")
  per_w = _BATCH // 32
  base = wid * per_w
  n_chunks = per_w // _CHUNK

  idx_bufs = ((idx_h0, idx_t0, idx_r0), (idx_h1, idx_t1, idx_r1))
  row_bufs = ((eh0, et0, rb0), (eh1, et1, rb1))
  sems = ((sem0, sem1, sem2), (sem3, sem4, sem5))
  phases = [(side, k) for side in range(2) for k in range(n_chunks)]

  def fire(i):
    side, k = phases[i]
    h_hbm = pos_h if side == 0 else neg_h
    t_hbm = pos_t if side == 0 else neg_t
    r_hbm = pos_r if side == 0 else neg_r
    ih, it, ir = idx_bufs[i % 2]
    eh, et, rb = row_bufs[i % 2]
    sm = sems[i % 2]
    b0 = base + k * _CHUNK
    pltpu.sync_copy(h_hbm.at[pl.ds(b0, _CHUNK)], ih)
    pltpu.sync_copy(t_hbm.at[pl.ds(b0, _CHUNK)], it)
    pltpu.sync_copy(r_hbm.at[pl.ds(b0, _CHUNK)], ir)
    return [
        pltpu.async_copy(ent.at[ih], eh, sm[0]),
        pltpu.async_copy(ent.at[it], et, sm[1]),
        pltpu.async_copy(rel.at[ir], rb, sm[2]),
    ]

  acc = jnp.zeros((_LANES,), jnp.float32)
  cps = fire(0)
  for i, (side, k) in enumerate(phases):
    for cp in cps:
      cp.wait()
    if i + 1 < len(phases):
      nxt = fire(i + 1)
    eh, et, rb = row_bufs[i % 2]
    sgn = -1.0 if side == 0 else 1.0

    def group8(g8, acc):
      for j in range(8):
        e = g8 * 8 + j
        s = jnp.zeros((_LANES,), jnp.float32)
        for c in range(_DIM // _LANES):
          lo = pl.ds(c * _LANES, _LANES)
          hi = pl.ds(_DIM + c * _LANES, _LANES)
          v1h = eh[e, lo]
          v2h = eh[e, hi]
          v1t = et[e, lo]
          v2t = et[e, hi]
          vr1 = rb[e, lo]
          vr2 = rb[e, hi]
          acc = acc + v1h * v1h + v2h * v2h + v1t * v1t
          acc = acc + v2t * v2t + vr1 * vr1 + vr2 * vr2
          s = s + (v1h * v1t + v2h * v2t) * vr1
          s = s + (v1h * v2t - v2h * v1t) * vr2
        score_buf[g8, pl.ds(j * _LANES, _LANES)] = sgn * s
      return acc

    acc = lax.fori_loop(0, _CHUNK // 8, group8, acc)
    row0 = (side * (_BATCH // 8) + wid * (per_w // 8) +
            k * (_CHUNK // 8))
    pltpu.sync_copy(score_buf, sc_out.at[pl.ds(row0, _CHUNK // 8)])
    if i + 1 < len(phases):
      cps = nxt

  acc_buf[...] = acc
  pltpu.sync_copy(acc_buf, sq_out.at[wid])


def _sc_call(ent, rel, pos_h, pos_t, pos_r, neg_h, neg_t, neg_r):
  mesh = plsc.VectorSubcoreMesh(core_axis_name="c", subcore_axis_name="s")
  f = pl.kernel(
      _sc_body,
      out_type=(
          jax.ShapeDtypeStruct((2 * _BATCH // 8, 8 * _LANES), jnp.float32),
          jax.ShapeDtypeStruct((32, _LANES), jnp.float32),
      ),
      mesh=mesh,
      scratch_types=[
          pltpu.VMEM((_CHUNK,), jnp.int32),
          pltpu.VMEM((_CHUNK,), jnp.int32),
          pltpu.VMEM((_CHUNK,), jnp.int32),
          pltpu.VMEM((_CHUNK,), jnp.int32),
          pltpu.VMEM((_CHUNK,), jnp.int32),
          pltpu.VMEM((_CHUNK,), jnp.int32),
          pltpu.VMEM((_CHUNK, 2 * _DIM), jnp.float32),
          pltpu.VMEM((_CHUNK, 2 * _DIM), jnp.float32),
          pltpu.VMEM((_CHUNK, 2 * _DIM), jnp.float32),
          pltpu.VMEM((_CHUNK, 2 * _DIM), jnp.float32),
          pltpu.VMEM((_CHUNK, 2 * _DIM), jnp.float32),
          pltpu.VMEM((_CHUNK, 2 * _DIM), jnp.float32),
          pltpu.VMEM((_CHUNK // 8, 8 * _LANES), jnp.float32),
          pltpu.VMEM((_LANES,), jnp.float32),
          pltpu.SemaphoreType.DMA,
          pltpu.SemaphoreType.DMA,
          pltpu.SemaphoreType.DMA,
          pltpu.SemaphoreType.DMA,
          pltpu.SemaphoreType.DMA,
          pltpu.SemaphoreType.DMA,
      ],
  )
  return f(ent, rel, pos_h, pos_t, pos_r, neg_h, neg_t, neg_r)


def _tc_body(s_ref, sq_ref, o_ref):
  x = s_ref[...]
  gsel = (lax.broadcasted_iota(jnp.int32, (8, 8 * _LANES), 1) // _LANES ==
          lax.broadcasted_iota(jnp.int32, (8, 8 * _LANES), 0)
          ).astype(jnp.float32)
  z = lax.dot_general(gsel, x, (((1,), (1,)), ((), ())),
                      preferred_element_type=jnp.float32)
  sp = jnp.maximum(z, 0.0) + jnp.log1p(jnp.exp(-jnp.abs(z)))
  loss = jnp.sum(sp) + _LAMBDA * jnp.sum(sq_ref[...])
  o_ref[0, 0] = loss


def _tc_call(scores, sq_part):
  return pl.pallas_call(
      _tc_body,
      out_shape=jax.ShapeDtypeStruct((1, 1), jnp.float32),
      out_specs=pl.BlockSpec(memory_space=pltpu.SMEM),
  )(scores, sq_part)


def _tr_body(e1t_ref, e2t_ref, o_ref):
  ident = (lax.broadcasted_iota(jnp.int32, (_DIM, _DIM), 0) ==
           lax.broadcasted_iota(jnp.int32, (_DIM, _DIM), 1)
           ).astype(jnp.bfloat16)
  dn = (((0,), (0,)), ((), ()))
  a = lax.dot_general(e1t_ref[...].astype(jnp.bfloat16), ident, dn,
                      preferred_element_type=jnp.float32)
  b = lax.dot_general(e2t_ref[...].astype(jnp.bfloat16), ident, dn,
                      preferred_element_type=jnp.float32)
  o_ref[...] = jnp.concatenate([a, b], axis=1)


def _transpose_concat(a_t, b_t, n_rows, blk):
  grid = (n_rows + blk - 1) // blk
  return pl.pallas_call(
      _tr_body,
      grid=(grid,),
      in_specs=[
          pl.BlockSpec((_DIM, blk), lambda i: (0, i)),
          pl.BlockSpec((_DIM, blk), lambda i: (0, i)),
      ],
      out_specs=pl.BlockSpec((blk, 2 * _DIM), lambda i: (i, 0)),
      out_shape=jax.ShapeDtypeStruct((n_rows, 2 * _DIM), jnp.float32),
  )(a_t, b_t)


@jax.jit
def kernel(pos_h, pos_t, pos_r, neg_h, neg_t, neg_r, ent1_embeddings,
           ent2_embeddings, rel1_embeddings, rel2_embeddings):
  ent = _transpose_concat(ent1_embeddings.T, ent2_embeddings.T,
                          1000000, 16384)
  rel = _transpose_concat(rel1_embeddings.T, rel2_embeddings.T,
                          1000, 1024)
  scores, sq_part = _sc_call(
      ent, rel,
      pos_h.astype(jnp.int32), pos_t.astype(jnp.int32),
      pos_r.astype(jnp.int32), neg_h.astype(jnp.int32),
      neg_t.astype(jnp.int32), neg_r.astype(jnp.int32))
  out = _tc_call(scores, sq_part)
  return out[0, 0]
